# baseline (speedup 1.0000x reference)
import functools

import jax
import jax.numpy as jnp
from jax import lax
from jax.experimental import pallas as pl
from jax.experimental.pallas import tpu as pltpu
from jax.experimental.pallas import tpu_sc as plsc

B, T, H, F, S, NE, NHT = 32, 512, 128, 16, 8, 64, 128
WORD, TAG, HID, WIDTH, MAXPOS = 21128, 512, 384, 16, 4
D_IN = 768

_BM = 512
_NBLK = (WORD + _BM - 1) // _BM


def _proj_body(w_ref, k_ref, b_ref, o_ref):
    o_ref[...] = (
        jnp.dot(w_ref[...], k_ref[...], preferred_element_type=jnp.float32)
        + b_ref[...]
    )


def _project_table(w_text, dense_kernel, dense_bias):
    return pl.pallas_call(
        _proj_body,
        grid=(_NBLK,),
        in_specs=[
            pl.BlockSpec((_BM, D_IN), lambda i: (i, 0)),
            pl.BlockSpec((D_IN, HID), lambda i: (0, 0)),
            pl.BlockSpec((1, HID), lambda i: (0, 0)),
        ],
        out_specs=pl.BlockSpec((_BM, HID), lambda i: (i, 0)),
        out_shape=jax.ShapeDtypeStruct((WORD, HID), jnp.float32),
    )(w_text, dense_kernel, dense_bias.reshape(1, HID))


_TEXT_CHUNK = 128


def _make_gather_fh():
    mesh = plsc.VectorSubcoreMesh(core_axis_name="c", subcore_axis_name="s")

    @functools.partial(
        pl.kernel,
        out_type=(
            jax.ShapeDtypeStruct((B * F, D_IN), jnp.float32),
            jax.ShapeDtypeStruct((B * H, HID), jnp.float32),
            jax.ShapeDtypeStruct((B, H, T), jnp.float32),
            jax.ShapeDtypeStruct((B, H, H), jnp.int32),
        ),
        mesh=mesh,
        scratch_types=(
            pltpu.VMEM((F,), jnp.int32),
            pltpu.VMEM((H,), jnp.int32),
            pltpu.VMEM((F, D_IN), jnp.float32),
            pltpu.VMEM((H // 2, HID), jnp.float32),
            pltpu.VMEM((NHT,), jnp.int32),
            pltpu.VMEM((NHT,), jnp.int32),
            pltpu.VMEM((NE,), jnp.int32),
            pltpu.VMEM((NE,), jnp.int32),
            pltpu.VMEM((NE,), jnp.int32),
            pltpu.VMEM((16,), jnp.int32),
            pltpu.VMEM((H, T), jnp.float32),
            pltpu.VMEM((H, H), jnp.int32),
            pltpu.SemaphoreType.DMA,
        ),
        compiler_params=pltpu.CompilerParams(needs_layout_passes=False),
    )
    def gather_fh(wt_hbm, wh_hbm,
                  fidx_hbm, hidx_hbm, hth_hbm, htt_hbm,
                  src_hbm, dst_hbm, typ_hbm,
                  field_out, html_out, h2t_out, h2h_out,
                  fidx_v, hidx_v, frows_v, hrows_v, hth_v, htt_v,
                  src_v, dst_v, typ_v, kbuf_v, h2t_v, h2h_v, sem):
        wid = lax.axis_index("s") * 2 + lax.axis_index("c")
        lane = lax.iota(jnp.int32, 16)

        pltpu.sync_copy(fidx_hbm.at[wid], fidx_v)
        pltpu.sync_copy(hidx_hbm.at[wid], hidx_v)
        pltpu.sync_copy(hth_hbm.at[wid], hth_v)
        pltpu.sync_copy(htt_hbm.at[wid], htt_v)
        pltpu.sync_copy(src_hbm.at[wid], src_v)
        pltpu.sync_copy(dst_hbm.at[wid], dst_v)
        pltpu.sync_copy(typ_hbm.at[wid], typ_v)

        pltpu.async_copy(wt_hbm.at[fidx_v], frows_v, sem).wait()
        pltpu.sync_copy(frows_v, field_out.at[pl.ds(wid * F, F)])

        ones16 = jnp.full((16,), 1.0, jnp.float32)

        def _ones_row(i, carry):
            for j in range(T // 16):
                h2t_v[i, pl.ds(j * 16, 16)] = ones16
            return carry

        lax.fori_loop(0, H, _ones_row, 0)
        zeros16 = jnp.zeros((16,), jnp.float32)
        for g in range(NHT // 16):
            hvec = hth_v[pl.ds(g * 16, 16)]
            tvec = htt_v[pl.ds(g * 16, 16)]
            plsc.store_scatter(h2t_v, [hvec, tvec], zeros16)
        pltpu.sync_copy(h2t_v, h2t_out.at[wid])

        zi16 = jnp.zeros((16,), jnp.int32)

        def _zero_row(i, carry):
            for j in range(H // 16):
                h2h_v[i, pl.ds(j * 16, 16)] = zi16
            return carry

        lax.fori_loop(0, H, _zero_row, 0)
        for g in range(NE // 16):
            s = src_v[pl.ds(g * 16, 16)]
            d = dst_v[pl.ds(g * 16, 16)]
            key2 = (s * H + d) * 16 + lane
            sk = lax.sort(key2)
            kbuf_v[...] = sk
            knext = plsc.load_gather(kbuf_v, [jnp.minimum(lane + 1, 15)])
            keep = ((knext // 16) != (sk // 16)) | (lane == 15)
            glane = (sk % 16) + g * 16
            tv = plsc.load_gather(typ_v, [glane])
            plsc.store_scatter(h2h_v, [sk // (16 * H), (sk // 16) % H],
                               tv, mask=keep)
        pltpu.sync_copy(h2h_v, h2h_out.at[wid])

        for c in range(2):
            pltpu.async_copy(
                wh_hbm.at[hidx_v.at[pl.ds(c * (H // 2), H // 2)]],
                hrows_v, sem).wait()
            pltpu.sync_copy(
                hrows_v,
                html_out.at[pl.ds(wid * H + c * (H // 2), H // 2)])

    return gather_fh


def _make_gather_text():
    mesh = plsc.VectorSubcoreMesh(core_axis_name="c", subcore_axis_name="s")

    @functools.partial(
        pl.kernel,
        out_type=jax.ShapeDtypeStruct((B * T, HID), jnp.float32),
        mesh=mesh,
        scratch_types=(
            pltpu.VMEM((T // _TEXT_CHUNK, _TEXT_CHUNK), jnp.int32),
            pltpu.VMEM((_TEXT_CHUNK, HID), jnp.float32),
            pltpu.VMEM((_TEXT_CHUNK, HID), jnp.float32),
            pltpu.SemaphoreType.DMA,
            pltpu.SemaphoreType.DMA,
        ),
    )
    def gather_text(p_hbm, tidx_hbm, text_out,
                    tidx_v, rows0_v, rows1_v, sem0, sem1):
        wid = lax.axis_index("s") * 2 + lax.axis_index("c")
        pltpu.sync_copy(tidx_hbm.at[wid], tidx_v)
        bufs = (rows0_v, rows1_v)
        sems = (sem0, sem1)
        nch = T // _TEXT_CHUNK
        copies = []
        for j in range(nch):
            copies.append(
                pltpu.async_copy(p_hbm.at[tidx_v.at[j]], bufs[j % 2],
                                 sems[j % 2]))
            if j > 0:
                copies[j - 1].wait()
                pltpu.sync_copy(
                    bufs[(j - 1) % 2],
                    text_out.at[pl.ds(wid * T + (j - 1) * _TEXT_CHUNK,
                                      _TEXT_CHUNK)],
                )
        copies[nch - 1].wait()
        pltpu.sync_copy(
            bufs[(nch - 1) % 2],
            text_out.at[pl.ds(wid * T + (nch - 1) * _TEXT_CHUNK,
                              _TEXT_CHUNK)],
        )

    return gather_text


_gather_cache = {}


def _gather_fh_fn(*args):
    if "fh" not in _gather_cache:
        _gather_cache["fh"] = _make_gather_fh()
    return _gather_cache["fh"](*args)


def _gather_text_fn(*args):
    if "text" not in _gather_cache:
        _gather_cache["text"] = _make_gather_text()
    return _gather_cache["text"](*args)


_TROWS = 256


def _t2t_body(seq_ref, hl_ref, out_ref, t2h_ref):
    r = pl.program_id(1)
    ll = lax.broadcasted_iota(jnp.int32, (1, T), 1)
    segl = jnp.zeros((1, T), jnp.int32)
    for j in range(S):
        sj = seq_ref[0, 0, j]
        segl = segl + (sj < ll).astype(jnp.int32)
    last = seq_ref[0, 0, S - 1]
    validl = (segl >= 1) & (ll <= last)
    cl = jnp.where(validl, (segl * 1024 + ll).astype(jnp.float32), 1.0e6)
    ckc = jnp.where(validl, (segl * 1024 + ll).astype(jnp.float32), -1.0e6)
    ckt = jnp.transpose(ckc)
    ck = jnp.where(r == 0, ckt[:_TROWS, :], ckt[_TROWS:, :])
    d = jnp.abs(ck - cl)
    out_ref[0] = jnp.where(d <= jnp.float32(WIDTH), 0.0, 1.0)

    @pl.when(r == 0)
    def _():
        t2h_ref[0] = hl_ref[0] == 0


def _t2t_mask(sequencelist, htmllist):
    t2t, t2h = pl.pallas_call(
        _t2t_body,
        grid=(B, T // _TROWS),
        in_specs=[
            pl.BlockSpec((1, 1, S), lambda b, r: (b, 0, 0),
                         memory_space=pltpu.SMEM),
            pl.BlockSpec((1, 1, H), lambda b, r: (b, 0, 0)),
        ],
        out_specs=[
            pl.BlockSpec((1, _TROWS, T), lambda b, r: (b, r, 0)),
            pl.BlockSpec((1, 1, H), lambda b, r: (b, 0, 0)),
        ],
        out_shape=[
            jax.ShapeDtypeStruct((B, T, T), jnp.float32),
            jax.ShapeDtypeStruct((B, 1, H), jnp.bool_),
        ],
        compiler_params=pltpu.CompilerParams(
            dimension_semantics=("parallel", "arbitrary")),
    )(sequencelist.reshape(B, 1, S), htmllist.reshape(B, 1, H))
    return t2t, t2h.reshape(B, H)


def kernel(fieldlist, textlist, htmllist, html_edge_src, html_edge_type,
           html_edge_dst, ht_html, ht_text, sequencelist, w_html, w_text,
           position_embedding, htmledge_embedding, dense_kernel, dense_bias):
    i32 = jnp.int32
    tidx = textlist.astype(i32).reshape(B, T // _TEXT_CHUNK, _TEXT_CHUNK)
    fidx = fieldlist.astype(i32)
    hidx = htmllist.astype(i32)

    field_rows, html_rows, h2t, h2h = _gather_fh_fn(
        w_text, w_html, fidx, hidx,
        ht_html.astype(i32), ht_text.astype(i32),
        html_edge_src.astype(i32), html_edge_dst.astype(i32),
        html_edge_type.astype(i32))
    p_table = _project_table(w_text, dense_kernel, dense_bias)
    text_rows = _gather_text_fn(p_table, tidx)

    field_embeds = field_rows.reshape(B, F, D_IN)
    text_embeds = text_rows.reshape(B, T, HID)
    html_embeds = html_rows.reshape(B, H, HID)

    t2t, t2h = _t2t_mask(sequencelist.astype(i32), htmllist.astype(i32))

    htmledge_complete = jnp.concatenate(
        [jnp.ones((1, HID), jnp.float32), htmledge_embedding], axis=0)

    return (field_embeds, text_embeds, html_embeds, t2t, h2h, h2t, t2h,
            position_embedding, htmledge_complete)


# speedup vs baseline: 1.0875x; 1.0875x over previous
import functools

import jax
import jax.numpy as jnp
from jax import lax
from jax.experimental import pallas as pl
from jax.experimental.pallas import tpu as pltpu
from jax.experimental.pallas import tpu_sc as plsc

B, T, H, F, S, NE, NHT = 32, 512, 128, 16, 8, 64, 128
WORD, TAG, HID, WIDTH, MAXPOS = 21128, 512, 384, 16, 4
D_IN = 768

_BM = 512
_NBLK = (WORD + _BM - 1) // _BM


def _proj_body(w_ref, k_ref, b_ref, o_ref):
    o_ref[...] = (
        jnp.dot(w_ref[...], k_ref[...], preferred_element_type=jnp.float32)
        + b_ref[...]
    )


def _project_table(w_text, dense_kernel, dense_bias):
    return pl.pallas_call(
        _proj_body,
        grid=(_NBLK,),
        in_specs=[
            pl.BlockSpec((_BM, D_IN), lambda i: (i, 0)),
            pl.BlockSpec((D_IN, HID), lambda i: (0, 0)),
            pl.BlockSpec((1, HID), lambda i: (0, 0)),
        ],
        out_specs=pl.BlockSpec((_BM, HID), lambda i: (i, 0)),
        out_shape=jax.ShapeDtypeStruct((WORD, HID), jnp.float32),
    )(w_text, dense_kernel, dense_bias.reshape(1, HID))


_TEXT_CHUNK = 128


def _make_gather_fh():
    mesh = plsc.VectorSubcoreMesh(core_axis_name="c", subcore_axis_name="s")

    @functools.partial(
        pl.kernel,
        out_type=(
            jax.ShapeDtypeStruct((B * F, D_IN), jnp.float32),
            jax.ShapeDtypeStruct((B * H, HID), jnp.float32),
            jax.ShapeDtypeStruct((B, H, T), jnp.float32),
            jax.ShapeDtypeStruct((B, H, H), jnp.int32),
        ),
        mesh=mesh,
        scratch_types=(
            pltpu.VMEM((F,), jnp.int32),
            pltpu.VMEM((H,), jnp.int32),
            pltpu.VMEM((F, D_IN), jnp.float32),
            pltpu.VMEM((H // 2, HID), jnp.float32),
            pltpu.VMEM((NHT,), jnp.int32),
            pltpu.VMEM((NHT,), jnp.int32),
            pltpu.VMEM((NE,), jnp.int32),
            pltpu.VMEM((NE,), jnp.int32),
            pltpu.VMEM((NE,), jnp.int32),
            pltpu.VMEM((16,), jnp.int32),
            pltpu.VMEM((H, T), jnp.float32),
            pltpu.VMEM((H, H), jnp.int32),
            pltpu.SemaphoreType.DMA,
        ),
        compiler_params=pltpu.CompilerParams(needs_layout_passes=False),
    )
    def gather_fh(wt_hbm, wh_hbm,
                  fidx_hbm, hidx_hbm, hth_hbm, htt_hbm,
                  src_hbm, dst_hbm, typ_hbm,
                  field_out, html_out, h2t_out, h2h_out,
                  fidx_v, hidx_v, frows_v, hrows_v, hth_v, htt_v,
                  src_v, dst_v, typ_v, kbuf_v, h2t_v, h2h_v, sem):
        wid = lax.axis_index("s") * 2 + lax.axis_index("c")
        lane = lax.iota(jnp.int32, 16)

        pltpu.sync_copy(fidx_hbm.at[wid], fidx_v)
        pltpu.sync_copy(hidx_hbm.at[wid], hidx_v)
        pltpu.sync_copy(hth_hbm.at[wid], hth_v)
        pltpu.sync_copy(htt_hbm.at[wid], htt_v)
        pltpu.sync_copy(src_hbm.at[wid], src_v)
        pltpu.sync_copy(dst_hbm.at[wid], dst_v)
        pltpu.sync_copy(typ_hbm.at[wid], typ_v)

        pltpu.async_copy(wt_hbm.at[fidx_v], frows_v, sem).wait()
        pltpu.sync_copy(frows_v, field_out.at[pl.ds(wid * F, F)])

        ones16 = jnp.full((16,), 1.0, jnp.float32)

        def _ones_row(i, carry):
            for j in range(T // 16):
                h2t_v[i, pl.ds(j * 16, 16)] = ones16
            return carry

        lax.fori_loop(0, H, _ones_row, 0)
        zeros16 = jnp.zeros((16,), jnp.float32)
        for g in range(NHT // 16):
            hvec = hth_v[pl.ds(g * 16, 16)]
            tvec = htt_v[pl.ds(g * 16, 16)]
            plsc.store_scatter(h2t_v, [hvec, tvec], zeros16)
        pltpu.sync_copy(h2t_v, h2t_out.at[wid])

        zi16 = jnp.zeros((16,), jnp.int32)

        def _zero_row(i, carry):
            for j in range(H // 16):
                h2h_v[i, pl.ds(j * 16, 16)] = zi16
            return carry

        lax.fori_loop(0, H, _zero_row, 0)
        for g in range(NE // 16):
            s = src_v[pl.ds(g * 16, 16)]
            d = dst_v[pl.ds(g * 16, 16)]
            key2 = (s * H + d) * 16 + lane
            sk = lax.sort(key2)
            kbuf_v[...] = sk
            knext = plsc.load_gather(kbuf_v, [jnp.minimum(lane + 1, 15)])
            keep = ((knext // 16) != (sk // 16)) | (lane == 15)
            glane = (sk % 16) + g * 16
            tv = plsc.load_gather(typ_v, [glane])
            plsc.store_scatter(h2h_v, [sk // (16 * H), (sk // 16) % H],
                               tv, mask=keep)
        pltpu.sync_copy(h2h_v, h2h_out.at[wid])

        for c in range(2):
            pltpu.async_copy(
                wh_hbm.at[hidx_v.at[pl.ds(c * (H // 2), H // 2)]],
                hrows_v, sem).wait()
            pltpu.sync_copy(
                hrows_v,
                html_out.at[pl.ds(wid * H + c * (H // 2), H // 2)])

    return gather_fh


def _make_gather_text():
    mesh = plsc.VectorSubcoreMesh(core_axis_name="c", subcore_axis_name="s")

    @functools.partial(
        pl.kernel,
        out_type=jax.ShapeDtypeStruct((B * T, HID), jnp.float32),
        mesh=mesh,
        scratch_types=(
            pltpu.VMEM((T // _TEXT_CHUNK, _TEXT_CHUNK), jnp.int32),
            pltpu.VMEM((_TEXT_CHUNK, HID), jnp.float32),
            pltpu.VMEM((_TEXT_CHUNK, HID), jnp.float32),
            pltpu.SemaphoreType.DMA,
            pltpu.SemaphoreType.DMA,
        ),
    )
    def gather_text(p_hbm, tidx_hbm, text_out,
                    tidx_v, rows0_v, rows1_v, sem0, sem1):
        wid = lax.axis_index("s") * 2 + lax.axis_index("c")
        pltpu.sync_copy(tidx_hbm.at[wid], tidx_v)
        bufs = (rows0_v, rows1_v)
        sems = (sem0, sem1)
        nch = T // _TEXT_CHUNK
        copies = []
        for j in range(nch):
            copies.append(
                pltpu.async_copy(p_hbm.at[tidx_v.at[j]], bufs[j % 2],
                                 sems[j % 2]))
            if j > 0:
                copies[j - 1].wait()
                pltpu.sync_copy(
                    bufs[(j - 1) % 2],
                    text_out.at[pl.ds(wid * T + (j - 1) * _TEXT_CHUNK,
                                      _TEXT_CHUNK)],
                )
        copies[nch - 1].wait()
        pltpu.sync_copy(
            bufs[(nch - 1) % 2],
            text_out.at[pl.ds(wid * T + (nch - 1) * _TEXT_CHUNK,
                              _TEXT_CHUNK)],
        )

    return gather_text


_gather_cache = {}


def _gather_fh_fn(*args):
    if "fh" not in _gather_cache:
        _gather_cache["fh"] = _make_gather_fh()
    return _gather_cache["fh"](*args)


def _gather_text_fn(*args):
    if "text" not in _gather_cache:
        _gather_cache["text"] = _make_gather_text()
    return _gather_cache["text"](*args)


def _t2t_body(seq_ref, hl_ref, out_ref, t2h_ref):
    ll = lax.broadcasted_iota(jnp.int32, (1, T), 1)
    segl = jnp.zeros((1, T), jnp.int32)
    for j in range(S):
        sj = seq_ref[0, 0, j]
        segl = segl + (sj < ll).astype(jnp.int32)
    last = seq_ref[0, 0, S - 1]
    validl = (segl >= 1) & (ll <= last)
    cl = jnp.where(validl, (segl * 1024 + ll).astype(jnp.float32), 1.0e6)
    ckc = jnp.where(validl, (segl * 1024 + ll).astype(jnp.float32), -1.0e6)
    ck = jnp.transpose(ckc)
    d = jnp.abs(ck - cl)
    out_ref[0] = jnp.where(d <= jnp.float32(WIDTH), 0.0, 1.0)
    t2h_ref[0] = hl_ref[0] == 0


def _t2t_mask(sequencelist, htmllist):
    t2t, t2h = pl.pallas_call(
        _t2t_body,
        grid=(B,),
        in_specs=[
            pl.BlockSpec((1, 1, S), lambda b: (b, 0, 0),
                         memory_space=pltpu.SMEM),
            pl.BlockSpec((1, 1, H), lambda b: (b, 0, 0)),
        ],
        out_specs=[
            pl.BlockSpec((1, T, T), lambda b: (b, 0, 0)),
            pl.BlockSpec((1, 1, H), lambda b: (b, 0, 0)),
        ],
        out_shape=[
            jax.ShapeDtypeStruct((B, T, T), jnp.float32),
            jax.ShapeDtypeStruct((B, 1, H), jnp.bool_),
        ],
        compiler_params=pltpu.CompilerParams(
            dimension_semantics=("parallel",)),
    )(sequencelist.reshape(B, 1, S), htmllist.reshape(B, 1, H))
    return t2t, t2h.reshape(B, H)


def kernel(fieldlist, textlist, htmllist, html_edge_src, html_edge_type,
           html_edge_dst, ht_html, ht_text, sequencelist, w_html, w_text,
           position_embedding, htmledge_embedding, dense_kernel, dense_bias):
    i32 = jnp.int32
    tidx = textlist.astype(i32).reshape(B, T // _TEXT_CHUNK, _TEXT_CHUNK)
    fidx = fieldlist.astype(i32)
    hidx = htmllist.astype(i32)

    field_rows, html_rows, h2t, h2h = _gather_fh_fn(
        w_text, w_html, fidx, hidx,
        ht_html.astype(i32), ht_text.astype(i32),
        html_edge_src.astype(i32), html_edge_dst.astype(i32),
        html_edge_type.astype(i32))
    p_table = _project_table(w_text, dense_kernel, dense_bias)
    text_rows = _gather_text_fn(p_table, tidx)

    field_embeds = field_rows.reshape(B, F, D_IN)
    text_embeds = text_rows.reshape(B, T, HID)
    html_embeds = html_rows.reshape(B, H, HID)

    t2t, t2h = _t2t_mask(sequencelist.astype(i32), htmllist.astype(i32))

    htmledge_complete = jnp.concatenate(
        [jnp.ones((1, HID), jnp.float32), htmledge_embedding], axis=0)

    return (field_embeds, text_embeds, html_embeds, t2t, h2h, h2t, t2h,
            position_embedding, htmledge_complete)


# speedup vs baseline: 1.2153x; 1.1175x over previous
import functools

import jax
import jax.numpy as jnp
from jax import lax
from jax.experimental import pallas as pl
from jax.experimental.pallas import tpu as pltpu
from jax.experimental.pallas import tpu_sc as plsc

B, T, H, F, S, NE, NHT = 32, 512, 128, 16, 8, 64, 128
WORD, TAG, HID, WIDTH, MAXPOS = 21128, 512, 384, 16, 4
D_IN = 768

_BM = 680
_PROWS = _BM * B


def _proj_t2t_body(w_ref, k_ref, b_ref, seq_ref, hl_ref,
                   p_ref, t2t_ref, t2h_ref):
    p_ref[...] = (
        jnp.dot(w_ref[...], k_ref[...], preferred_element_type=jnp.float32)
        + b_ref[...]
    )
    ll = lax.broadcasted_iota(jnp.int32, (1, T), 1)
    segl = jnp.zeros((1, T), jnp.int32)
    for j in range(S):
        sj = seq_ref[0, 0, j]
        segl = segl + (sj < ll).astype(jnp.int32)
    last = seq_ref[0, 0, S - 1]
    validl = (segl >= 1) & (ll <= last)
    cl = jnp.where(validl, (segl * 1024 + ll).astype(jnp.float32), 1.0e6)
    ckc = jnp.where(validl, (segl * 1024 + ll).astype(jnp.float32), -1.0e6)
    ck = jnp.transpose(ckc)
    d = jnp.abs(ck - cl)
    t2t_ref[0] = jnp.where(d <= jnp.float32(WIDTH), 0.0, 1.0)
    t2h_ref[0] = hl_ref[0] == 0


def _project_and_t2t(w_text, dense_kernel, dense_bias, sequencelist,
                     htmllist):
    return pl.pallas_call(
        _proj_t2t_body,
        grid=(B,),
        in_specs=[
            pl.BlockSpec((_BM, D_IN), lambda i: (i, 0)),
            pl.BlockSpec((D_IN, HID), lambda i: (0, 0)),
            pl.BlockSpec((1, HID), lambda i: (0, 0)),
            pl.BlockSpec((1, 1, S), lambda i: (i, 0, 0),
                         memory_space=pltpu.SMEM),
            pl.BlockSpec((1, 1, H), lambda i: (i, 0, 0)),
        ],
        out_specs=[
            pl.BlockSpec((_BM, HID), lambda i: (i, 0)),
            pl.BlockSpec((1, T, T), lambda i: (i, 0, 0)),
            pl.BlockSpec((1, 1, H), lambda i: (i, 0, 0)),
        ],
        out_shape=[
            jax.ShapeDtypeStruct((_PROWS, HID), jnp.float32),
            jax.ShapeDtypeStruct((B, T, T), jnp.float32),
            jax.ShapeDtypeStruct((B, 1, H), jnp.bool_),
        ],
    )(w_text, dense_kernel, dense_bias.reshape(1, HID),
      sequencelist.reshape(B, 1, S), htmllist.reshape(B, 1, H))


_TEXT_CHUNK = 128


def _make_gather_fh():
    mesh = plsc.VectorSubcoreMesh(core_axis_name="c", subcore_axis_name="s")

    @functools.partial(
        pl.kernel,
        out_type=(
            jax.ShapeDtypeStruct((B * F, D_IN), jnp.float32),
            jax.ShapeDtypeStruct((B * H, HID), jnp.float32),
            jax.ShapeDtypeStruct((B, H, T), jnp.float32),
            jax.ShapeDtypeStruct((B, H, H), jnp.int32),
        ),
        mesh=mesh,
        scratch_types=(
            pltpu.VMEM((F,), jnp.int32),
            pltpu.VMEM((H,), jnp.int32),
            pltpu.VMEM((F, D_IN), jnp.float32),
            pltpu.VMEM((H // 2, HID), jnp.float32),
            pltpu.VMEM((NHT,), jnp.int32),
            pltpu.VMEM((NHT,), jnp.int32),
            pltpu.VMEM((NE,), jnp.int32),
            pltpu.VMEM((NE,), jnp.int32),
            pltpu.VMEM((NE,), jnp.int32),
            pltpu.VMEM((16,), jnp.int32),
            pltpu.VMEM((H, T), jnp.float32),
            pltpu.VMEM((H, H), jnp.int32),
            pltpu.SemaphoreType.DMA,
        ),
        compiler_params=pltpu.CompilerParams(needs_layout_passes=False),
    )
    def gather_fh(wt_hbm, wh_hbm,
                  fidx_hbm, hidx_hbm, hth_hbm, htt_hbm,
                  src_hbm, dst_hbm, typ_hbm,
                  field_out, html_out, h2t_out, h2h_out,
                  fidx_v, hidx_v, frows_v, hrows_v, hth_v, htt_v,
                  src_v, dst_v, typ_v, kbuf_v, h2t_v, h2h_v, sem):
        wid = lax.axis_index("s") * 2 + lax.axis_index("c")
        lane = lax.iota(jnp.int32, 16)

        pltpu.sync_copy(fidx_hbm.at[wid], fidx_v)
        pltpu.sync_copy(hidx_hbm.at[wid], hidx_v)
        pltpu.sync_copy(hth_hbm.at[wid], hth_v)
        pltpu.sync_copy(htt_hbm.at[wid], htt_v)
        pltpu.sync_copy(src_hbm.at[wid], src_v)
        pltpu.sync_copy(dst_hbm.at[wid], dst_v)
        pltpu.sync_copy(typ_hbm.at[wid], typ_v)

        pltpu.async_copy(wt_hbm.at[fidx_v], frows_v, sem).wait()
        pltpu.sync_copy(frows_v, field_out.at[pl.ds(wid * F, F)])

        ones16 = jnp.full((16,), 1.0, jnp.float32)

        def _ones_row(i, carry):
            for j in range(T // 16):
                h2t_v[i, pl.ds(j * 16, 16)] = ones16
            return carry

        lax.fori_loop(0, H, _ones_row, 0)
        zeros16 = jnp.zeros((16,), jnp.float32)
        for g in range(NHT // 16):
            hvec = hth_v[pl.ds(g * 16, 16)]
            tvec = htt_v[pl.ds(g * 16, 16)]
            plsc.store_scatter(h2t_v, [hvec, tvec], zeros16)
        pltpu.sync_copy(h2t_v, h2t_out.at[wid])

        zi16 = jnp.zeros((16,), jnp.int32)

        def _zero_row(i, carry):
            for j in range(H // 16):
                h2h_v[i, pl.ds(j * 16, 16)] = zi16
            return carry

        lax.fori_loop(0, H, _zero_row, 0)
        for g in range(NE // 16):
            s = src_v[pl.ds(g * 16, 16)]
            d = dst_v[pl.ds(g * 16, 16)]
            key2 = (s * H + d) * 16 + lane
            sk = lax.sort(key2)
            kbuf_v[...] = sk
            knext = plsc.load_gather(kbuf_v, [jnp.minimum(lane + 1, 15)])
            keep = ((knext // 16) != (sk // 16)) | (lane == 15)
            glane = (sk % 16) + g * 16
            tv = plsc.load_gather(typ_v, [glane])
            plsc.store_scatter(h2h_v, [sk // (16 * H), (sk // 16) % H],
                               tv, mask=keep)
        pltpu.sync_copy(h2h_v, h2h_out.at[wid])

        for c in range(2):
            pltpu.async_copy(
                wh_hbm.at[hidx_v.at[pl.ds(c * (H // 2), H // 2)]],
                hrows_v, sem).wait()
            pltpu.sync_copy(
                hrows_v,
                html_out.at[pl.ds(wid * H + c * (H // 2), H // 2)])

    return gather_fh


def _make_gather_text():
    mesh = plsc.VectorSubcoreMesh(core_axis_name="c", subcore_axis_name="s")

    @functools.partial(
        pl.kernel,
        out_type=jax.ShapeDtypeStruct((B * T, HID), jnp.float32),
        mesh=mesh,
        scratch_types=(
            pltpu.VMEM((T // _TEXT_CHUNK, _TEXT_CHUNK), jnp.int32),
            pltpu.VMEM((_TEXT_CHUNK, HID), jnp.float32),
            pltpu.VMEM((_TEXT_CHUNK, HID), jnp.float32),
            pltpu.SemaphoreType.DMA,
            pltpu.SemaphoreType.DMA,
        ),
    )
    def gather_text(p_hbm, tidx_hbm, text_out,
                    tidx_v, rows0_v, rows1_v, sem0, sem1):
        wid = lax.axis_index("s") * 2 + lax.axis_index("c")
        pltpu.sync_copy(tidx_hbm.at[wid], tidx_v)
        bufs = (rows0_v, rows1_v)
        sems = (sem0, sem1)
        nch = T // _TEXT_CHUNK
        copies = []
        for j in range(nch):
            copies.append(
                pltpu.async_copy(p_hbm.at[tidx_v.at[j]], bufs[j % 2],
                                 sems[j % 2]))
            if j > 0:
                copies[j - 1].wait()
                pltpu.sync_copy(
                    bufs[(j - 1) % 2],
                    text_out.at[pl.ds(wid * T + (j - 1) * _TEXT_CHUNK,
                                      _TEXT_CHUNK)],
                )
        copies[nch - 1].wait()
        pltpu.sync_copy(
            bufs[(nch - 1) % 2],
            text_out.at[pl.ds(wid * T + (nch - 1) * _TEXT_CHUNK,
                              _TEXT_CHUNK)],
        )

    return gather_text


_gather_cache = {}


def _gather_fh_fn(*args):
    if "fh" not in _gather_cache:
        _gather_cache["fh"] = _make_gather_fh()
    return _gather_cache["fh"](*args)


def _gather_text_fn(*args):
    if "text" not in _gather_cache:
        _gather_cache["text"] = _make_gather_text()
    return _gather_cache["text"](*args)


def kernel(fieldlist, textlist, htmllist, html_edge_src, html_edge_type,
           html_edge_dst, ht_html, ht_text, sequencelist, w_html, w_text,
           position_embedding, htmledge_embedding, dense_kernel, dense_bias):
    i32 = jnp.int32
    tidx = textlist.astype(i32).reshape(B, T // _TEXT_CHUNK, _TEXT_CHUNK)
    fidx = fieldlist.astype(i32)
    hidx = htmllist.astype(i32)

    field_rows, html_rows, h2t, h2h = _gather_fh_fn(
        w_text, w_html, fidx, hidx,
        ht_html.astype(i32), ht_text.astype(i32),
        html_edge_src.astype(i32), html_edge_dst.astype(i32),
        html_edge_type.astype(i32))
    p_table, t2t, t2h3 = _project_and_t2t(
        w_text, dense_kernel, dense_bias,
        sequencelist.astype(i32), htmllist.astype(i32))
    t2h = t2h3.reshape(B, H)
    text_rows = _gather_text_fn(p_table, tidx)

    field_embeds = field_rows.reshape(B, F, D_IN)
    text_embeds = text_rows.reshape(B, T, HID)
    html_embeds = html_rows.reshape(B, H, HID)

    htmledge_complete = jnp.concatenate(
        [jnp.ones((1, HID), jnp.float32), htmledge_embedding], axis=0)

    return (field_embeds, text_embeds, html_embeds, t2t, h2h, h2t, t2h,
            position_embedding, htmledge_complete)
